# initial kernel scaffold (unmeasured)
import jax
import jax.numpy as jnp
from jax import lax
from jax.experimental import pallas as pl
from jax.experimental.pallas import tpu as pltpu


def kernel(
    x,
):
    def body(*refs):
        pass

    out_shape = jax.ShapeDtypeStruct(..., jnp.float32)
    return pl.pallas_call(body, out_shape=out_shape)(...)


# baseline (device time: 54218 ns/iter reference)
import jax
import jax.numpy as jnp
from jax import lax
from jax.experimental import pallas as pl
from jax.experimental.pallas import tpu as pltpu


def kernel(x):
    _, m, n = x.shape
    half = n // 2

    def body(x_ref, out_ref, comm_ref, send_sem, recv_sem):
        my_x = lax.axis_index("x")
        my_y = lax.axis_index("y")
        my_z = lax.axis_index("z")
        peer_y = 1 - my_y

        barrier_sem = pltpu.get_barrier_semaphore()
        pl.semaphore_signal(
            barrier_sem,
            inc=1,
            device_id=(my_x, peer_y, my_z),
            device_id_type=pl.DeviceIdType.MESH,
        )
        pl.semaphore_wait(barrier_sem, 1)

        rdma = pltpu.make_async_remote_copy(
            src_ref=x_ref.at[0, :, pl.ds(peer_y * half, half)],
            dst_ref=comm_ref,
            send_sem=send_sem,
            recv_sem=recv_sem,
            device_id=(my_x, peer_y, my_z),
            device_id_type=pl.DeviceIdType.MESH,
        )
        rdma.start()
        rdma.wait()

        out_ref[:, :] = x_ref[0, :, pl.ds(my_y * half, half)] + comm_ref[:, :]

    return pl.pallas_call(
        body,
        out_shape=jax.ShapeDtypeStruct((m, half), jnp.float32),
        in_specs=[pl.BlockSpec(memory_space=pltpu.VMEM)],
        out_specs=pl.BlockSpec(memory_space=pltpu.VMEM),
        scratch_shapes=[
            pltpu.VMEM((m, half), jnp.float32),
            pltpu.SemaphoreType.DMA,
            pltpu.SemaphoreType.DMA,
        ],
        compiler_params=pltpu.CompilerParams(collective_id=0),
    )(x)


# device time: 36869 ns/iter; 1.4706x vs baseline; 1.4706x over previous
import jax
import jax.numpy as jnp
from jax import lax
from jax.experimental import pallas as pl
from jax.experimental.pallas import tpu as pltpu

C = 8


def kernel(x):
    _, m, n = x.shape
    half = n // 2
    rows_half = m // 2
    ck = rows_half // C

    def body(x_ref, out_ref, recv_y_ref, recv_x_ref,
             ysend_sems, yrecv_sems, xsend_sems, xrecv_sems):
        my_x = lax.axis_index("x")
        my_y = lax.axis_index("y")
        my_z = lax.axis_index("z")
        peer = (my_x, 1 - my_y, my_z)
        xnbr = (1 - my_x, my_y, my_z)

        barrier_sem = pltpu.get_barrier_semaphore()
        for nbr in (peer, xnbr):
            pl.semaphore_signal(
                barrier_sem, inc=1,
                device_id=nbr, device_id_type=pl.DeviceIdType.MESH,
            )
        pl.semaphore_wait(barrier_sem, 2)

        col0 = (1 - my_y) * half
        row0 = my_x * rows_half

        def y_rdma(c):
            return pltpu.make_async_remote_copy(
                src_ref=x_ref.at[0, pl.ds(row0 + c * ck, ck), pl.ds(col0, half)],
                dst_ref=recv_y_ref.at[pl.ds(c * ck, ck), :],
                send_sem=ysend_sems.at[c],
                recv_sem=yrecv_sems.at[c],
                device_id=peer,
                device_id_type=pl.DeviceIdType.MESH,
            )

        def x_rdma(c):
            return pltpu.make_async_remote_copy(
                src_ref=recv_y_ref.at[pl.ds(c * ck, ck), :],
                dst_ref=recv_x_ref.at[pl.ds(c * ck, ck), :],
                send_sem=xsend_sems.at[c],
                recv_sem=xrecv_sems.at[c],
                device_id=xnbr,
                device_id_type=pl.DeviceIdType.MESH,
            )

        for c in range(C):
            y_rdma(c).start()

        mycol = my_y * half
        for c in range(C):
            y_rdma(c).wait_recv()
            x_rdma(c).start()
            g = row0 + c * ck
            out_ref[pl.ds(g, ck), :] = (
                x_ref[0, pl.ds(g, ck), pl.ds(mycol, half)]
                + recv_y_ref[pl.ds(c * ck, ck), :]
            )

        other0 = (1 - my_x) * rows_half
        for c in range(C):
            x_rdma(c).wait_recv()
            g = other0 + c * ck
            out_ref[pl.ds(g, ck), :] = (
                x_ref[0, pl.ds(g, ck), pl.ds(mycol, half)]
                + recv_x_ref[pl.ds(c * ck, ck), :]
            )

        for c in range(C):
            y_rdma(c).wait_send()
            x_rdma(c).wait_send()

    return pl.pallas_call(
        body,
        out_shape=jax.ShapeDtypeStruct((m, half), jnp.float32),
        in_specs=[pl.BlockSpec(memory_space=pltpu.VMEM)],
        out_specs=pl.BlockSpec(memory_space=pltpu.VMEM),
        scratch_shapes=[
            pltpu.VMEM((rows_half, half), jnp.float32),
            pltpu.VMEM((rows_half, half), jnp.float32),
            pltpu.SemaphoreType.DMA((C,)),
            pltpu.SemaphoreType.DMA((C,)),
            pltpu.SemaphoreType.DMA((C,)),
            pltpu.SemaphoreType.DMA((C,)),
        ],
        compiler_params=pltpu.CompilerParams(collective_id=0),
    )(x)
